# Initial kernel scaffold; baseline (speedup 1.0000x reference)
import jax
import jax.numpy as jnp
from jax.experimental import pallas as pl


def kernel(x, edge_index, Wl1, bl1, Wr1, Wl2, bl2, Wr2):
    raise NotImplementedError("write your pallas kernel here")


# speedup vs baseline: 6.4059x; 6.4059x over previous
import functools

import jax
import jax.numpy as jnp
from jax import lax
from jax.experimental import pallas as pl
from jax.experimental.pallas import tpu as pltpu
from jax.experimental.pallas import tpu_sc as plsc

N = 10000
E = 320000
D = 128
NPAD = 10240
NC = 2
NS = 16
NW = NC * NS
EPT = E // NW
C = 80
NCHUNK = EPT // C
RPT = NPAD // NS
BR = 1024

_MESH = plsc.VectorSubcoreMesh(core_axis_name="c", subcore_axis_name="s")


@functools.partial(
    pl.kernel,
    out_type=(
        jax.ShapeDtypeStruct((NC, NPAD, D), jnp.float32),
        jax.ShapeDtypeStruct((NC, NPAD), jnp.float32),
    ),
    mesh=_MESH,
    scratch_types=[
        pltpu.VMEM_SHARED((NPAD, D), jnp.float32),
        pltpu.VMEM_SHARED((NPAD,), jnp.float32),
        pltpu.VMEM((NCHUNK, C), jnp.int32),
        pltpu.VMEM((NCHUNK, C), jnp.int32),
        pltpu.VMEM((C, D), jnp.float32),
        pltpu.VMEM((C,), jnp.float32),
        pltpu.SemaphoreType.DMA,
    ],
)
def _aggregate(x_hbm, sidx_hbm, didx_hbm, z2d_hbm, z1d_hbm, ones_hbm,
               p_hbm, cnt_hbm,
               acc_sh, cnt_sh, sidx_v, didx_v, rows_v, ones_v, sem):
    cid = lax.axis_index("c")
    sid = lax.axis_index("s")
    wid = cid * NS + sid
    rbase = sid * RPT

    pltpu.sync_copy(ones_hbm, ones_v)
    pltpu.sync_copy(sidx_hbm.at[wid], sidx_v)
    pltpu.sync_copy(didx_hbm.at[wid], didx_v)

    @pl.loop(0, RPT // 128)
    def _(k):
        pltpu.sync_copy(z2d_hbm, acc_sh.at[pl.ds(rbase + k * 128, 128)])
    pltpu.sync_copy(z1d_hbm, cnt_sh.at[pl.ds(rbase, RPT)])
    plsc.subcore_barrier()

    @pl.loop(0, NCHUNK)
    def _(j):
        pltpu.async_copy(x_hbm.at[sidx_v.at[j]], rows_v, sem).wait()
        pltpu.sync_copy(rows_v, acc_sh.at[didx_v.at[j]], add=True)
        pltpu.sync_copy(ones_v, cnt_sh.at[didx_v.at[j]], add=True)
    plsc.subcore_barrier()

    pltpu.sync_copy(acc_sh.at[pl.ds(rbase, RPT)],
                    p_hbm.at[cid, pl.ds(rbase, RPT)])
    pltpu.sync_copy(cnt_sh.at[pl.ds(rbase, RPT)],
                    cnt_hbm.at[cid, pl.ds(rbase, RPT)])


def _dense_body(p0, p1, c0, c1, xb, wl, bl, wr, o, *, relu):
    cnt = jnp.maximum(c0[...] + c1[...], 1.0)
    mean = (p0[...] + p1[...]) / cnt
    acc = lax.dot_general(mean, wl[...], (((1,), (1,)), ((), ())),
                          preferred_element_type=jnp.float32)
    acc = acc + bl[...]
    acc = acc + lax.dot_general(xb[...], wr[...], (((1,), (1,)), ((), ())),
                                preferred_element_type=jnp.float32)
    o[...] = jnp.maximum(acc, 0.0) if relu else acc


def _dense(relu):
    row = lambda i: (i, 0)
    fixed = lambda i: (0, 0)
    return pl.pallas_call(
        functools.partial(_dense_body, relu=relu),
        grid=(NPAD // BR,),
        in_specs=[
            pl.BlockSpec((BR, D), row),
            pl.BlockSpec((BR, D), row),
            pl.BlockSpec((BR, 1), row),
            pl.BlockSpec((BR, 1), row),
            pl.BlockSpec((BR, D), row),
            pl.BlockSpec((D, D), fixed),
            pl.BlockSpec((1, D), fixed),
            pl.BlockSpec((D, D), fixed),
        ],
        out_specs=pl.BlockSpec((BR, D), row),
        out_shape=jax.ShapeDtypeStruct((NPAD, D), jnp.float32),
    )


def _layer(x_pad, sidx3, didx3, z2d, z1d, ones_c, Wl, bl, Wr, relu):
    p, cnt = _aggregate(x_pad, sidx3, didx3, z2d, z1d, ones_c)
    return _dense(relu)(
        p[0], p[1], cnt[0][:, None], cnt[1][:, None], x_pad,
        Wl, bl.reshape(1, D), Wr)


def kernel(x, edge_index, Wl1, bl1, Wr1, Wl2, bl2, Wr2):
    src = edge_index[0].astype(jnp.int32)
    dst = edge_index[1].astype(jnp.int32)
    sidx3 = src.reshape(NW, NCHUNK, C)
    didx3 = dst.reshape(NW, NCHUNK, C)
    x_pad = jnp.pad(x, ((0, NPAD - N), (0, 0)))
    z2d = jnp.zeros((128, D), jnp.float32)
    z1d = jnp.zeros((RPT,), jnp.float32)
    ones_c = jnp.ones((C,), jnp.float32)

    h = _layer(x_pad, sidx3, didx3, z2d, z1d, ones_c, Wl1, bl1, Wr1, True)
    out = _layer(h, sidx3, didx3, z2d, z1d, ones_c, Wl2, bl2, Wr2, False)
    return out[:N]


# speedup vs baseline: 8.8454x; 1.3808x over previous
import functools

import jax
import jax.numpy as jnp
from jax import lax
from jax.experimental import pallas as pl
from jax.experimental.pallas import tpu as pltpu
from jax.experimental.pallas import tpu_sc as plsc

N = 10000
E = 320000
D = 128
NPAD = 10240
NC = 2
NS = 16
NW = NC * NS
EPT = E // NW
C = 80
NCHUNK = EPT // C
RPT = NPAD // NS
BR = 1024

_MESH = plsc.VectorSubcoreMesh(core_axis_name="c", subcore_axis_name="s")


def _make_aggregate(with_counts):
    out_type = jax.ShapeDtypeStruct((NC, NPAD, D), jnp.float32)
    if with_counts:
        out_type = (out_type, jax.ShapeDtypeStruct((NC, NPAD), jnp.float32))
    scratch = [
        pltpu.VMEM_SHARED((NPAD, D), jnp.float32),
        pltpu.VMEM((NCHUNK, C), jnp.int32),
        pltpu.VMEM((1, C), jnp.int32),
        pltpu.VMEM((1, C), jnp.int32),
        pltpu.VMEM((C, D), jnp.float32),
        pltpu.VMEM((C, D), jnp.float32),
    ]
    scratch += [pltpu.SemaphoreType.DMA] * 6
    if with_counts:
        scratch += [
            pltpu.VMEM_SHARED((NPAD,), jnp.float32),
            pltpu.VMEM((C,), jnp.float32),
            pltpu.SemaphoreType.DMA,
            pltpu.SemaphoreType.DMA,
        ]

    def body(x_hbm, sidx_hbm, didx_hbm, z2d_hbm, z1d_hbm, ones_hbm,
             *rest):
        if with_counts:
            (p_hbm, cnt_hbm, acc_sh, sidx_v, didx0, didx1, rows0, rows1,
             sg0, sg1, ss0, ss1, sd0, sd1, cnt_sh, ones_v, sc0, sc1) = rest
        else:
            (p_hbm, acc_sh, sidx_v, didx0, didx1, rows0, rows1,
             sg0, sg1, ss0, ss1, sd0, sd1) = rest
        cid = lax.axis_index("c")
        sid = lax.axis_index("s")
        wid = cid * NS + sid
        rbase = sid * RPT

        pltpu.sync_copy(sidx_hbm.at[wid], sidx_v)
        if with_counts:
            pltpu.sync_copy(ones_hbm, ones_v)
            pltpu.sync_copy(z1d_hbm, cnt_sh.at[pl.ds(rbase, RPT)])

        @pl.loop(0, RPT // 128)
        def _(k):
            pltpu.sync_copy(z2d_hbm, acc_sh.at[pl.ds(rbase + k * 128, 128)])
        plsc.subcore_barrier()

        def fire_gather(j, rows, sem):
            pltpu.async_copy(x_hbm.at[sidx_v.at[j]], rows, sem)

        def wait_gather(j, rows, sem):
            pltpu.make_async_copy(x_hbm.at[sidx_v.at[j]], rows, sem).wait()

        def fire_didx(j, didx, sem):
            pltpu.async_copy(didx_hbm.at[wid, pl.ds(j, 1)], didx, sem)

        def wait_didx(j, didx, sem):
            pltpu.make_async_copy(didx_hbm.at[wid, pl.ds(j, 1)], didx,
                                  sem).wait()

        def fire_counts(didx, sem):
            if with_counts:
                pltpu.async_copy(ones_v, cnt_sh.at[didx.at[0]], sem,
                                 add=True)

        def wait_counts(didx, sem):
            if with_counts:
                pltpu.make_async_copy(ones_v, cnt_sh.at[didx.at[0]],
                                      sem).wait()

        pltpu.sync_copy(didx_hbm.at[wid, pl.ds(0, 1)], didx0)
        fire_gather(0, rows0, sg0)
        wait_gather(0, rows0, sg0)
        pltpu.sync_copy(rows0, acc_sh.at[didx0.at[0]], add=True)
        fire_counts(didx0, sc0 if with_counts else None)
        wait_counts(didx0, sc0 if with_counts else None)

        fire_didx(1, didx0, sd0)
        fire_didx(2, didx1, sd1)
        fire_gather(1, rows0, sg0)
        fire_gather(2, rows1, sg1)

        @pl.loop(0, (NCHUNK - 3) // 2)
        def _(t):
            j = 1 + 2 * t
            wait_gather(j, rows0, sg0)
            wait_didx(j, didx0, sd0)
            pltpu.async_copy(rows0, acc_sh.at[didx0.at[0]], ss0, add=True)
            fire_counts(didx0, sc0 if with_counts else None)
            wait_gather(j + 1, rows1, sg1)
            wait_didx(j + 1, didx1, sd1)
            pltpu.async_copy(rows1, acc_sh.at[didx1.at[0]], ss1, add=True)
            fire_counts(didx1, sc1 if with_counts else None)
            pltpu.make_async_copy(rows0, acc_sh.at[didx0.at[0]], ss0).wait()
            wait_counts(didx0, sc0 if with_counts else None)
            fire_didx(j + 2, didx0, sd0)
            fire_gather(j + 2, rows0, sg0)
            pltpu.make_async_copy(rows1, acc_sh.at[didx1.at[0]], ss1).wait()
            wait_counts(didx1, sc1 if with_counts else None)
            fire_didx(j + 3, didx1, sd1)
            fire_gather(j + 3, rows1, sg1)

        wait_gather(NCHUNK - 2, rows0, sg0)
        wait_didx(NCHUNK - 2, didx0, sd0)
        pltpu.sync_copy(rows0, acc_sh.at[didx0.at[0]], add=True)
        fire_counts(didx0, sc0 if with_counts else None)
        wait_gather(NCHUNK - 1, rows1, sg1)
        wait_didx(NCHUNK - 1, didx1, sd1)
        pltpu.sync_copy(rows1, acc_sh.at[didx1.at[0]], add=True)
        fire_counts(didx1, sc1 if with_counts else None)
        wait_counts(didx0, sc0 if with_counts else None)
        wait_counts(didx1, sc1 if with_counts else None)
        plsc.subcore_barrier()

        pltpu.sync_copy(acc_sh.at[pl.ds(rbase, RPT)],
                        p_hbm.at[cid, pl.ds(rbase, RPT)])
        if with_counts:
            pltpu.sync_copy(cnt_sh.at[pl.ds(rbase, RPT)],
                            cnt_hbm.at[cid, pl.ds(rbase, RPT)])

    return pl.kernel(body, out_type=out_type, mesh=_MESH,
                     scratch_types=scratch)


_agg_counts = _make_aggregate(True)
_agg_plain = _make_aggregate(False)


def _dense_body(p0, p1, c0, c1, xb, wl, bl, wr, o, *, relu):
    cnt = jnp.maximum(c0[...] + c1[...], 1.0)
    mean = (p0[...] + p1[...]) / cnt
    acc = lax.dot_general(mean, wl[...], (((1,), (1,)), ((), ())),
                          preferred_element_type=jnp.float32)
    acc = acc + bl[...]
    acc = acc + lax.dot_general(xb[...], wr[...], (((1,), (1,)), ((), ())),
                                preferred_element_type=jnp.float32)
    o[...] = jnp.maximum(acc, 0.0) if relu else acc


def _dense(relu):
    row = lambda i: (i, 0)
    fixed = lambda i: (0, 0)
    return pl.pallas_call(
        functools.partial(_dense_body, relu=relu),
        grid=(NPAD // BR,),
        in_specs=[
            pl.BlockSpec((BR, D), row),
            pl.BlockSpec((BR, D), row),
            pl.BlockSpec((BR, 1), row),
            pl.BlockSpec((BR, 1), row),
            pl.BlockSpec((BR, D), row),
            pl.BlockSpec((D, D), fixed),
            pl.BlockSpec((1, D), fixed),
            pl.BlockSpec((D, D), fixed),
        ],
        out_specs=pl.BlockSpec((BR, D), row),
        out_shape=jax.ShapeDtypeStruct((NPAD, D), jnp.float32),
    )


def kernel(x, edge_index, Wl1, bl1, Wr1, Wl2, bl2, Wr2):
    src = edge_index[0].astype(jnp.int32)
    dst = edge_index[1].astype(jnp.int32)
    sidx3 = src.reshape(NW, NCHUNK, C)
    didx3 = dst.reshape(NW, NCHUNK, C)
    x_pad = jnp.pad(x, ((0, NPAD - N), (0, 0)))
    z2d = jnp.zeros((128, D), jnp.float32)
    z1d = jnp.zeros((RPT,), jnp.float32)
    ones_c = jnp.ones((C,), jnp.float32)

    p1, cnt = _agg_counts(x_pad, sidx3, didx3, z2d, z1d, ones_c)
    c0 = cnt[0][:, None]
    c1 = cnt[1][:, None]
    h = _dense(True)(p1[0], p1[1], c0, c1, x_pad,
                     Wl1, bl1.reshape(1, D), Wr1)
    p2 = _agg_plain(h, sidx3, didx3, z2d, z1d, ones_c)
    out = _dense(False)(p2[0], p2[1], c0, c1, h,
                        Wl2, bl2.reshape(1, D), Wr2)
    return out[:N]


# speedup vs baseline: 9.1616x; 1.0358x over previous
import functools

import jax
import jax.numpy as jnp
from jax import lax
from jax.experimental import pallas as pl
from jax.experimental.pallas import tpu as pltpu
from jax.experimental.pallas import tpu_sc as plsc

N = 10000
E = 320000
D = 128
NPAD = 10240
NC = 2
NS = 16
NW = NC * NS
EPT = E // NW
C = 80
NCHUNK = EPT // C
RPT = NPAD // NS
BR = 1024

_MESH = plsc.VectorSubcoreMesh(core_axis_name="c", subcore_axis_name="s")


def _make_aggregate(with_counts):
    out_type = jax.ShapeDtypeStruct((NC, NPAD, D), jnp.float32)
    if with_counts:
        out_type = (out_type, jax.ShapeDtypeStruct((NC, NPAD), jnp.float32))
    scratch = [
        pltpu.VMEM_SHARED((NPAD, D), jnp.float32),
        pltpu.VMEM((NCHUNK, C), jnp.int32),
        pltpu.VMEM((1, C), jnp.int32),
        pltpu.VMEM((1, C), jnp.int32),
        pltpu.VMEM((C, D), jnp.float32),
        pltpu.VMEM((C, D), jnp.float32),
    ]
    scratch += [pltpu.SemaphoreType.DMA] * 6
    if with_counts:
        scratch += [
            pltpu.VMEM_SHARED((NPAD,), jnp.float32),
            pltpu.VMEM((C,), jnp.float32),
            pltpu.SemaphoreType.DMA,
            pltpu.SemaphoreType.DMA,
        ]

    def body(x_hbm, sidx_hbm, didx_hbm, z2d_hbm, z1d_hbm, ones_hbm,
             *rest):
        if with_counts:
            (p_hbm, cnt_hbm, acc_sh, sidx_v, didx0, didx1, rows0, rows1,
             sg0, sg1, ss0, ss1, sd0, sd1, cnt_sh, ones_v, sc0, sc1) = rest
        else:
            (p_hbm, acc_sh, sidx_v, didx0, didx1, rows0, rows1,
             sg0, sg1, ss0, ss1, sd0, sd1) = rest
        cid = lax.axis_index("c")
        sid = lax.axis_index("s")
        wid = cid * NS + sid
        rbase = sid * RPT

        pltpu.sync_copy(sidx_hbm.at[wid], sidx_v)
        if with_counts:
            pltpu.sync_copy(ones_hbm, ones_v)
            pltpu.sync_copy(z1d_hbm, cnt_sh.at[pl.ds(rbase, RPT)])

        @pl.loop(0, RPT // 128)
        def _(k):
            pltpu.sync_copy(z2d_hbm, acc_sh.at[pl.ds(rbase + k * 128, 128)])
        plsc.subcore_barrier()

        def fire_gather(j, rows, sem):
            pltpu.async_copy(x_hbm.at[sidx_v.at[j]], rows, sem)

        def wait_gather(j, rows, sem):
            pltpu.make_async_copy(x_hbm.at[sidx_v.at[j]], rows, sem).wait()

        def fire_didx(j, didx, sem):
            pltpu.async_copy(didx_hbm.at[wid, pl.ds(j, 1)], didx, sem)

        def wait_didx(j, didx, sem):
            pltpu.make_async_copy(didx_hbm.at[wid, pl.ds(j, 1)], didx,
                                  sem).wait()

        def fire_counts(didx, sem):
            if with_counts:
                pltpu.async_copy(ones_v, cnt_sh.at[didx.at[0]], sem,
                                 add=True)

        def wait_counts(didx, sem):
            if with_counts:
                pltpu.make_async_copy(ones_v, cnt_sh.at[didx.at[0]],
                                      sem).wait()

        pltpu.sync_copy(didx_hbm.at[wid, pl.ds(0, 1)], didx0)
        fire_gather(0, rows0, sg0)
        wait_gather(0, rows0, sg0)
        pltpu.sync_copy(rows0, acc_sh.at[didx0.at[0]], add=True)
        fire_counts(didx0, sc0 if with_counts else None)
        wait_counts(didx0, sc0 if with_counts else None)

        fire_didx(1, didx0, sd0)
        fire_didx(2, didx1, sd1)
        fire_gather(1, rows0, sg0)
        fire_gather(2, rows1, sg1)

        @pl.loop(0, (NCHUNK - 3) // 2)
        def _(t):
            j = 1 + 2 * t
            wait_gather(j, rows0, sg0)
            wait_didx(j, didx0, sd0)
            pltpu.async_copy(rows0, acc_sh.at[didx0.at[0]], ss0, add=True)
            fire_counts(didx0, sc0 if with_counts else None)
            wait_gather(j + 1, rows1, sg1)
            wait_didx(j + 1, didx1, sd1)
            pltpu.async_copy(rows1, acc_sh.at[didx1.at[0]], ss1, add=True)
            fire_counts(didx1, sc1 if with_counts else None)
            pltpu.make_async_copy(rows0, acc_sh.at[didx0.at[0]], ss0).wait()
            wait_counts(didx0, sc0 if with_counts else None)
            fire_didx(j + 2, didx0, sd0)
            fire_gather(j + 2, rows0, sg0)
            pltpu.make_async_copy(rows1, acc_sh.at[didx1.at[0]], ss1).wait()
            wait_counts(didx1, sc1 if with_counts else None)
            fire_didx(j + 3, didx1, sd1)
            fire_gather(j + 3, rows1, sg1)

        wait_gather(NCHUNK - 2, rows0, sg0)
        wait_didx(NCHUNK - 2, didx0, sd0)
        pltpu.sync_copy(rows0, acc_sh.at[didx0.at[0]], add=True)
        fire_counts(didx0, sc0 if with_counts else None)
        wait_gather(NCHUNK - 1, rows1, sg1)
        wait_didx(NCHUNK - 1, didx1, sd1)
        pltpu.sync_copy(rows1, acc_sh.at[didx1.at[0]], add=True)
        fire_counts(didx1, sc1 if with_counts else None)
        wait_counts(didx0, sc0 if with_counts else None)
        wait_counts(didx1, sc1 if with_counts else None)
        plsc.subcore_barrier()

        pltpu.sync_copy(acc_sh.at[pl.ds(rbase, RPT)],
                        p_hbm.at[cid, pl.ds(rbase, RPT)])
        if with_counts:
            pltpu.sync_copy(cnt_sh.at[pl.ds(rbase, RPT)],
                            cnt_hbm.at[cid, pl.ds(rbase, RPT)])

    return pl.kernel(body, out_type=out_type, mesh=_MESH,
                     scratch_types=scratch)


_agg_counts = _make_aggregate(True)
_agg_plain = _make_aggregate(False)


def _dense_body(pa, pb, ca, cb, xb, wl, bl, wr, o, *, relu):
    cnt = jnp.maximum(ca[0] + cb[0], 1.0)
    mean = (pa[0] + pb[0]) / cnt
    acc = lax.dot_general(mean, wl[...], (((1,), (1,)), ((), ())),
                          preferred_element_type=jnp.float32)
    acc = acc + bl[...]
    acc = acc + lax.dot_general(xb[...], wr[...], (((1,), (1,)), ((), ())),
                                preferred_element_type=jnp.float32)
    o[...] = jnp.maximum(acc, 0.0) if relu else acc


def _dense(relu):
    return pl.pallas_call(
        functools.partial(_dense_body, relu=relu),
        grid=(NPAD // BR,),
        in_specs=[
            pl.BlockSpec((1, BR, D), lambda i: (0, i, 0)),
            pl.BlockSpec((1, BR, D), lambda i: (1, i, 0)),
            pl.BlockSpec((1, BR, 1), lambda i: (0, i, 0)),
            pl.BlockSpec((1, BR, 1), lambda i: (1, i, 0)),
            pl.BlockSpec((BR, D), lambda i: (i, 0)),
            pl.BlockSpec((D, D), lambda i: (0, 0)),
            pl.BlockSpec((1, D), lambda i: (0, 0)),
            pl.BlockSpec((D, D), lambda i: (0, 0)),
        ],
        out_specs=pl.BlockSpec((BR, D), lambda i: (i, 0)),
        out_shape=jax.ShapeDtypeStruct((NPAD, D), jnp.float32),
    )


def kernel(x, edge_index, Wl1, bl1, Wr1, Wl2, bl2, Wr2):
    src = edge_index[0].astype(jnp.int32)
    dst = edge_index[1].astype(jnp.int32)
    sidx3 = src.reshape(NW, NCHUNK, C)
    didx3 = dst.reshape(NW, NCHUNK, C)
    x_pad = jnp.pad(x, ((0, NPAD - N), (0, 0)))
    z2d = jnp.zeros((128, D), jnp.float32)
    z1d = jnp.zeros((RPT,), jnp.float32)
    ones_c = jnp.ones((C,), jnp.float32)

    p1, cnt = _agg_counts(x_pad, sidx3, didx3, z2d, z1d, ones_c)
    cnt3 = cnt.reshape(NC, NPAD, 1)
    h = _dense(True)(p1, p1, cnt3, cnt3, x_pad, Wl1, bl1.reshape(1, D), Wr1)
    p2 = _agg_plain(h, sidx3, didx3, z2d, z1d, ones_c)
    out = _dense(False)(p2, p2, cnt3, cnt3, h, Wl2, bl2.reshape(1, D), Wr2)
    return out[:N]


# speedup vs baseline: 10.6645x; 1.1640x over previous
import functools

import jax
import jax.numpy as jnp
from jax import lax
from jax.experimental import pallas as pl
from jax.experimental.pallas import tpu as pltpu
from jax.experimental.pallas import tpu_sc as plsc

N = 10000
E = 320000
D = 128
NPAD = 10112
NC = 2
NS = 16
NW = NC * NS
EPT = E // NW
C = 40
NCHUNK = EPT // C
NSLOT = 3
RPT = NPAD // NS
BR = 1264
CPAD = 10240
CRPT = CPAD // NS

_MESH = plsc.VectorSubcoreMesh(core_axis_name="c", subcore_axis_name="s")


def _make_aggregate(with_counts):
    out_type = jax.ShapeDtypeStruct((NC, NPAD, D), jnp.float32)
    if with_counts:
        out_type = (out_type, jax.ShapeDtypeStruct((NC, CPAD), jnp.float32))
    scratch = [
        pltpu.VMEM_SHARED((NPAD, D), jnp.float32),
        pltpu.VMEM((NCHUNK, C), jnp.int32),
    ]
    scratch += [pltpu.VMEM((1, C), jnp.int32)] * NSLOT
    scratch += [pltpu.VMEM((C, D), jnp.float32)] * NSLOT
    scratch += [pltpu.SemaphoreType.DMA] * (3 * NSLOT)
    if with_counts:
        scratch += [
            pltpu.VMEM_SHARED((CPAD,), jnp.float32),
            pltpu.VMEM((C,), jnp.float32),
        ]
        scratch += [pltpu.SemaphoreType.DMA] * NSLOT

    def body(x_hbm, sidx_hbm, didx_hbm, z2d_hbm, z1d_hbm, ones_hbm,
             *rest):
        if with_counts:
            p_hbm, cnt_hbm = rest[0], rest[1]
            rest = rest[2:]
        else:
            p_hbm = rest[0]
            rest = rest[1:]
        acc_sh, sidx_v = rest[0], rest[1]
        didx = rest[2:2 + NSLOT]
        rows = rest[2 + NSLOT:2 + 2 * NSLOT]
        sg = rest[2 + 2 * NSLOT:2 + 3 * NSLOT]
        sd = rest[2 + 3 * NSLOT:2 + 4 * NSLOT]
        ss = rest[2 + 4 * NSLOT:2 + 5 * NSLOT]
        if with_counts:
            cnt_sh, ones_v = rest[2 + 5 * NSLOT], rest[3 + 5 * NSLOT]
            sc = rest[4 + 5 * NSLOT:4 + 6 * NSLOT]
        cid = lax.axis_index("c")
        sid = lax.axis_index("s")
        wid = cid * NS + sid
        rbase = sid * RPT

        pltpu.sync_copy(sidx_hbm.at[wid], sidx_v)
        if with_counts:
            pltpu.sync_copy(ones_hbm, ones_v)
            pltpu.sync_copy(z1d_hbm, cnt_sh.at[pl.ds(sid * CRPT, CRPT)])

        @pl.loop(0, 4)
        def _(k):
            pltpu.sync_copy(z2d_hbm, acc_sh.at[pl.ds(rbase + k * 128, 128)])
        pltpu.sync_copy(z2d_hbm.at[pl.ds(0, RPT - 512)],
                        acc_sh.at[pl.ds(rbase + 512, RPT - 512)])
        plsc.subcore_barrier()

        def fire_gather(j, k):
            pltpu.async_copy(x_hbm.at[sidx_v.at[j]], rows[k], sg[k])

        def wait_gather(j, k):
            pltpu.make_async_copy(x_hbm.at[sidx_v.at[j]], rows[k],
                                  sg[k]).wait()

        def fire_didx(j, k):
            pltpu.async_copy(didx_hbm.at[wid, pl.ds(j, 1)], didx[k], sd[k])

        def wait_didx(j, k):
            pltpu.make_async_copy(didx_hbm.at[wid, pl.ds(j, 1)], didx[k],
                                  sd[k]).wait()

        for k in range(NSLOT):
            fire_didx(k, k)
            fire_gather(k, k)

        nloop = (NCHUNK + NSLOT - 1) // NSLOT

        @pl.loop(0, nloop)
        def _(t):
            for k in range(NSLOT):
                j = t * NSLOT + k

                @pl.when(j < NCHUNK)
                def _():
                    wait_gather(j, k)
                    wait_didx(j, k)
                    pltpu.async_copy(rows[k], acc_sh.at[didx[k].at[0]],
                                     ss[k], add=True)
                    if with_counts:
                        pltpu.async_copy(ones_v, cnt_sh.at[didx[k].at[0]],
                                         sc[k], add=True)

                @pl.when(j + NSLOT < NCHUNK)
                def _():
                    pltpu.make_async_copy(rows[k], acc_sh.at[didx[k].at[0]],
                                          ss[k]).wait()
                    if with_counts:
                        pltpu.make_async_copy(ones_v,
                                              cnt_sh.at[didx[k].at[0]],
                                              sc[k]).wait()
                    fire_didx(j + NSLOT, k)
                    fire_gather(j + NSLOT, k)

        for k in range(NSLOT):
            pltpu.make_async_copy(rows[k], acc_sh.at[didx[k].at[0]],
                                  ss[k]).wait()
            if with_counts:
                pltpu.make_async_copy(ones_v, cnt_sh.at[didx[k].at[0]],
                                      sc[k]).wait()
        plsc.subcore_barrier()

        pltpu.sync_copy(acc_sh.at[pl.ds(rbase, RPT)],
                        p_hbm.at[cid, pl.ds(rbase, RPT)])
        if with_counts:
            pltpu.sync_copy(cnt_sh.at[pl.ds(sid * CRPT, CRPT)],
                            cnt_hbm.at[cid, pl.ds(sid * CRPT, CRPT)])

    return pl.kernel(body, out_type=out_type, mesh=_MESH,
                     scratch_types=scratch)


_agg_counts = _make_aggregate(True)
_agg_plain = _make_aggregate(False)


def _dense_body(pa, pb, ca, cb, xb, wl, bl, wr, o, *, relu):
    cnt = jnp.maximum(ca[0] + cb[0], 1.0)
    mean = (pa[0] + pb[0]) / cnt
    acc = lax.dot_general(mean, wl[...], (((1,), (1,)), ((), ())),
                          preferred_element_type=jnp.float32)
    acc = acc + bl[...]
    acc = acc + lax.dot_general(xb[...], wr[...], (((1,), (1,)), ((), ())),
                                preferred_element_type=jnp.float32)
    o[...] = jnp.maximum(acc, 0.0) if relu else acc


def _dense(relu):
    return pl.pallas_call(
        functools.partial(_dense_body, relu=relu),
        grid=(NPAD // BR,),
        in_specs=[
            pl.BlockSpec((1, BR, D), lambda i: (0, i, 0)),
            pl.BlockSpec((1, BR, D), lambda i: (1, i, 0)),
            pl.BlockSpec((1, BR, 1), lambda i: (0, i, 0)),
            pl.BlockSpec((1, BR, 1), lambda i: (1, i, 0)),
            pl.BlockSpec((BR, D), lambda i: (i, 0)),
            pl.BlockSpec((D, D), lambda i: (0, 0)),
            pl.BlockSpec((1, D), lambda i: (0, 0)),
            pl.BlockSpec((D, D), lambda i: (0, 0)),
        ],
        out_specs=pl.BlockSpec((BR, D), lambda i: (i, 0)),
        out_shape=jax.ShapeDtypeStruct((NPAD, D), jnp.float32),
    )


def kernel(x, edge_index, Wl1, bl1, Wr1, Wl2, bl2, Wr2):
    src = edge_index[0].astype(jnp.int32)
    dst = edge_index[1].astype(jnp.int32)
    sidx3 = src.reshape(NW, NCHUNK, C)
    didx3 = dst.reshape(NW, NCHUNK, C)
    x_pad = jnp.pad(x, ((0, NPAD - N), (0, 0)))
    z2d = jnp.zeros((128, D), jnp.float32)
    z1d = jnp.zeros((CRPT,), jnp.float32)
    ones_c = jnp.ones((C,), jnp.float32)

    p1, cnt = _agg_counts(x_pad, sidx3, didx3, z2d, z1d, ones_c)
    cnt3 = cnt[:, :NPAD].reshape(NC, NPAD, 1)
    h = _dense(True)(p1, p1, cnt3, cnt3, x_pad, Wl1, bl1.reshape(1, D), Wr1)
    p2 = _agg_plain(h, sidx3, didx3, z2d, z1d, ones_c)
    out = _dense(False)(p2, p2, cnt3, cnt3, h, Wl2, bl2.reshape(1, D), Wr2)
    return out[:N]
